# baseline (speedup 1.0000x reference)
import functools

import jax
import jax.numpy as jnp
from jax import lax
from jax.experimental import pallas as pl
from jax.experimental.pallas import tpu as pltpu
from jax.experimental.pallas import tpu_sc as plsc

NUM_FIELDS = 26
FIELD_DIM = 100000
EMBED_DIM = 16
BATCH = 16384
BF = BATCH * NUM_FIELDS
BN_EPS = 1e-5

NC, NS = 2, 16
NW = NC * NS
PER_W = BF // NW
CH = 1664
NCH = PER_W // CH
SB = BATCH // NW

MLP_BLK = 1024


def _sc_body(idx_hbm, emb_hbm, lin_hbm, h_out, lin_out,
             idx_c, rows_v, idxl_v, vals_v, linb_v, sem_e, sem_l):
    w = lax.axis_index("s") * NC + lax.axis_index("c")
    base = w * PER_W

    pltpu.sync_copy(idx_hbm.at[pl.ds(base, PER_W)], idxl_v)
    lin_cp = pltpu.async_copy(lin_hbm.at[idxl_v], vals_v, sem_l)

    for c in range(NCH):
        pltpu.sync_copy(idx_hbm.at[pl.ds(base + c * CH, CH)], idx_c)
        pltpu.async_copy(emb_hbm.at[idx_c], rows_v, sem_e).wait()
        pltpu.sync_copy(rows_v, h_out.at[pl.ds(base + c * CH, CH)])

    lin_cp.wait()
    lane = lax.iota(jnp.int32, 16) * NUM_FIELDS

    def gbody(g, _):
        row0 = g * (16 * NUM_FIELDS)
        acc = plsc.load_gather(vals_v, [lane + row0])
        for f in range(1, NUM_FIELDS):
            acc = acc + plsc.load_gather(vals_v, [lane + (row0 + f)])
        linb_v[pl.ds(g * 16, 16)] = acc
        return 0

    lax.fori_loop(0, SB // 16, gbody, 0)
    pltpu.sync_copy(linb_v, lin_out.at[pl.ds(w * SB, SB)])


_sc_gather = functools.partial(
    pl.kernel,
    out_type=[
        jax.ShapeDtypeStruct((BF, 2, 8), jnp.float32),
        jax.ShapeDtypeStruct((BATCH,), jnp.float32),
    ],
    mesh=plsc.VectorSubcoreMesh(core_axis_name="c", subcore_axis_name="s"),
    scratch_types=[
        pltpu.VMEM((CH,), jnp.int32),
        pltpu.VMEM((CH, 2, 8), jnp.float32),
        pltpu.VMEM((PER_W,), jnp.int32),
        pltpu.VMEM((PER_W,), jnp.float32),
        pltpu.VMEM((SB,), jnp.float32),
        pltpu.SemaphoreType.DMA,
        pltpu.SemaphoreType.DMA,
    ],
    compiler_params=pltpu.CompilerParams(use_tc_tiling_on_sc=False,
                                         needs_layout_passes=False),
)(_sc_body)


def _mlp_body(h_ref, lin_ref, W1_ref, b1_ref, g1_ref, be1_ref,
              W2_ref, b2_ref, g2_ref, be2_ref, W3_ref, b3_ref, lb_ref,
              out_ref):
    inv = 1.0 / (1.0 + BN_EPS) ** 0.5
    s1 = g1_ref[...] * inv
    t1 = b1_ref[...] * s1 + be1_ref[...]
    a = jnp.dot(h_ref[...], W1_ref[...], preferred_element_type=jnp.float32)
    a = jnp.maximum(a * s1 + t1, 0.0)
    s2 = g2_ref[...] * inv
    t2 = b2_ref[...] * s2 + be2_ref[...]
    a = jnp.dot(a, W2_ref[...], preferred_element_type=jnp.float32)
    a = jnp.maximum(a * s2 + t2, 0.0)
    deep = jnp.sum(a * W3_ref[...], axis=1, keepdims=True)
    out_ref[...] = deep + b3_ref[...] + lb_ref[...] + lin_ref[...]


def _mlp(h2d, lin2d, W1, b1, g1, be1, W2, b2, g2, be2, W3r, b3, lbias):
    full = lambda shape: pl.BlockSpec(shape, lambda i: (0, 0))
    return pl.pallas_call(
        _mlp_body,
        grid=(BATCH // MLP_BLK,),
        in_specs=[
            pl.BlockSpec((MLP_BLK, NUM_FIELDS * EMBED_DIM), lambda i: (i, 0)),
            pl.BlockSpec((MLP_BLK, 1), lambda i: (i, 0)),
            full((NUM_FIELDS * EMBED_DIM, 256)),
            full((1, 256)), full((1, 256)), full((1, 256)),
            full((256, 128)),
            full((1, 128)), full((1, 128)), full((1, 128)),
            full((1, 128)),
            full((1, 1)), full((1, 1)),
        ],
        out_specs=pl.BlockSpec((MLP_BLK, 1), lambda i: (i, 0)),
        out_shape=jax.ShapeDtypeStruct((BATCH, 1), jnp.float32),
    )(h2d, lin2d, W1, b1, g1, be1, W2, b2, g2, be2, W3r, b3, lbias)


def kernel(x, emb_W, lin_W, lin_bias, W1, b1, g1, be1, W2, b2, g2, be2,
           W3, b3):
    offs = jnp.arange(NUM_FIELDS, dtype=jnp.int32) * FIELD_DIM
    idx = x.astype(jnp.int32) + offs[None, :]
    idx_emb = idx.reshape(-1)
    emb3 = emb_W.reshape(NUM_FIELDS * FIELD_DIM, 2, 8)
    h, lin_sum = _sc_gather(idx_emb, emb3, lin_W.reshape(-1))
    out = _mlp(
        h.reshape(BATCH, NUM_FIELDS * EMBED_DIM),
        lin_sum.reshape(BATCH, 1),
        W1, b1.reshape(1, -1), g1.reshape(1, -1), be1.reshape(1, -1),
        W2, b2.reshape(1, -1), g2.reshape(1, -1), be2.reshape(1, -1),
        W3.reshape(1, -1), b3.reshape(1, 1), lin_bias.reshape(1, 1),
    )
    return jnp.squeeze(out, axis=1)


# speedup vs baseline: 5.9101x; 5.9101x over previous
import functools

import jax
import jax.numpy as jnp
from jax import lax
from jax.experimental import pallas as pl
from jax.experimental.pallas import tpu as pltpu
from jax.experimental.pallas import tpu_sc as plsc

NUM_FIELDS = 26
FIELD_DIM = 100000
EMBED_DIM = 16
BATCH = 16384
BF = BATCH * NUM_FIELDS
BN_EPS = 1e-5

NC, NS = 2, 16
NW = NC * NS
PER_W = BF // NW
CH = 1664
NCH = PER_W // CH
SB = BATCH // NW

MLP_BLK = 1024


def _sc_body(idx_hbm, emb_hbm, lin_hbm, h_out, lin_out,
             idx_c, rows_v, idxl_v, vals_v, linb_v, sem_e, sem_l):
    w = lax.axis_index("s") * NC + lax.axis_index("c")
    base = w * PER_W

    pltpu.sync_copy(idx_hbm.at[pl.ds(base, PER_W)], idxl_v)
    lin_cp = pltpu.async_copy(lin_hbm.at[idxl_v], vals_v, sem_l)

    for c in range(NCH):
        pltpu.sync_copy(idx_hbm.at[pl.ds(base + c * CH, CH)], idx_c)
        pltpu.async_copy(emb_hbm.at[idx_c], rows_v, sem_e).wait()
        pltpu.sync_copy(rows_v, h_out.at[pl.ds(base + c * CH, CH)])

    lin_cp.wait()
    lane = lax.iota(jnp.int32, 16) * NUM_FIELDS

    def gbody(g, _):
        row0 = g * (16 * NUM_FIELDS)
        acc = plsc.load_gather(vals_v, [lane + row0])
        for f in range(1, NUM_FIELDS):
            acc = acc + plsc.load_gather(vals_v, [lane + (row0 + f)])
        linb_v[pl.ds(g * 16, 16)] = acc
        return 0

    lax.fori_loop(0, SB // 16, gbody, 0)
    pltpu.sync_copy(linb_v, lin_out.at[pl.ds(w * SB, SB)])


_sc_gather = functools.partial(
    pl.kernel,
    out_type=[
        jax.ShapeDtypeStruct((BF, EMBED_DIM), jnp.float32),
        jax.ShapeDtypeStruct((BATCH,), jnp.float32),
    ],
    mesh=plsc.VectorSubcoreMesh(core_axis_name="c", subcore_axis_name="s"),
    scratch_types=[
        pltpu.VMEM((CH,), jnp.int32),
        pltpu.VMEM((CH, EMBED_DIM), jnp.float32),
        pltpu.VMEM((PER_W,), jnp.int32),
        pltpu.VMEM((PER_W,), jnp.float32),
        pltpu.VMEM((SB,), jnp.float32),
        pltpu.SemaphoreType.DMA,
        pltpu.SemaphoreType.DMA,
    ],
    compiler_params=pltpu.CompilerParams(use_tc_tiling_on_sc=False,
                                         needs_layout_passes=False),
)(_sc_body)


TOTAL_VOCAB = NUM_FIELDS * FIELD_DIM
RW = 1024
NB_FULL = TOTAL_VOCAB // RW
TAIL = TOTAL_VOCAB - NB_FULL * RW
NSTEP = 80
OUT_WORDS = TOTAL_VOCAB * EMBED_DIM


def _rp_extract(inb, rowb, nrows):
    lane = lax.iota(jnp.int32, 16)
    U = 8

    def ebody(i, _):
        q0 = i * U
        for u in range(U):
            vals = plsc.load_gather(inb, [lane, jnp.full((16,), q0 + u,
                                                         jnp.int32)])
            rowb[pl.ds((q0 + u) * EMBED_DIM, 16)] = vals
        return 0

    lax.fori_loop(0, nrows // U, ebody, 0)


def _rp_body(embT_hbm, out_hbm, in0, in1, row0, row1, tail_v,
             si0, si1, so0, so1, st):
    w = lax.axis_index("s") * NC + lax.axis_index("c")
    start = (w * NB_FULL) // NW
    nb_max = NB_FULL - 1

    def jb_of(k):
        return jnp.minimum(start + k, nb_max)

    pltpu.async_copy(embT_hbm.at[:, pl.ds(jb_of(0) * RW, RW)], in0, si0)

    def step(t, _):
        for par, inb, si, si_nxt, inb_nxt, rowb, so in (
                (0, in0, si0, si1, in1, row0, so0),
                (1, in1, si1, si0, in0, row1, so1)):
            k = 2 * t + par
            pltpu.make_async_copy(embT_hbm.at[:, pl.ds(0, RW)], inb,
                                  si).wait()
            if par == 0:
                pltpu.async_copy(
                    embT_hbm.at[:, pl.ds(jb_of(k + 1) * RW, RW)],
                    inb_nxt, si_nxt)
            else:
                @pl.when(t < NSTEP // 2 - 1)
                def _():
                    pltpu.async_copy(
                        embT_hbm.at[:, pl.ds(jb_of(k + 1) * RW, RW)],
                        inb_nxt, si_nxt)

            @pl.when(t >= 1)
            def _():
                pltpu.make_async_copy(
                    row0, out_hbm.at[pl.ds(0, RW * EMBED_DIM)], so).wait()

            _rp_extract(inb, rowb, RW)
            pltpu.async_copy(
                rowb, out_hbm.at[pl.ds(jb_of(k) * (RW * EMBED_DIM),
                                       RW * EMBED_DIM)], so)
        return 0

    lax.fori_loop(0, NSTEP // 2, step, 0)
    pltpu.make_async_copy(row0, out_hbm.at[pl.ds(0, RW * EMBED_DIM)],
                          so0).wait()
    pltpu.make_async_copy(row1, out_hbm.at[pl.ds(0, RW * EMBED_DIM)],
                          so1).wait()

    @pl.when(w == NW - 1)
    def _():
        pltpu.sync_copy(embT_hbm.at[:, pl.ds(NB_FULL * RW, TAIL)], tail_v)
        lane = lax.iota(jnp.int32, 16)

        def tbody(q, _):
            vals = plsc.load_gather(tail_v, [lane, jnp.full((16,), q,
                                                            jnp.int32)])
            row0[pl.ds(q * EMBED_DIM, 16)] = vals
            return 0

        lax.fori_loop(0, TAIL, tbody, 0)
        pltpu.async_copy(
            row0.at[pl.ds(0, TAIL * EMBED_DIM)],
            out_hbm.at[pl.ds(NB_FULL * RW * EMBED_DIM, TAIL * EMBED_DIM)],
            st).wait()


_repack = functools.partial(
    pl.kernel,
    out_type=jax.ShapeDtypeStruct((OUT_WORDS,), jnp.float32),
    mesh=plsc.VectorSubcoreMesh(core_axis_name="c", subcore_axis_name="s"),
    scratch_types=[
        pltpu.VMEM((EMBED_DIM, RW), jnp.float32),
        pltpu.VMEM((EMBED_DIM, RW), jnp.float32),
        pltpu.VMEM((RW * EMBED_DIM,), jnp.float32),
        pltpu.VMEM((RW * EMBED_DIM,), jnp.float32),
        pltpu.VMEM((EMBED_DIM, TAIL), jnp.float32),
        pltpu.SemaphoreType.DMA,
        pltpu.SemaphoreType.DMA,
        pltpu.SemaphoreType.DMA,
        pltpu.SemaphoreType.DMA,
        pltpu.SemaphoreType.DMA,
    ],
    compiler_params=pltpu.CompilerParams(use_tc_tiling_on_sc=True,
                                         needs_layout_passes=False),
)(_rp_body)


def _mlp_body(h_ref, lin_ref, W1_ref, b1_ref, g1_ref, be1_ref,
              W2_ref, b2_ref, g2_ref, be2_ref, W3_ref, b3_ref, lb_ref,
              out_ref):
    inv = 1.0 / (1.0 + BN_EPS) ** 0.5
    s1 = g1_ref[...] * inv
    t1 = b1_ref[...] * s1 + be1_ref[...]
    a = jnp.dot(h_ref[...], W1_ref[...], preferred_element_type=jnp.float32)
    a = jnp.maximum(a * s1 + t1, 0.0)
    s2 = g2_ref[...] * inv
    t2 = b2_ref[...] * s2 + be2_ref[...]
    a = jnp.dot(a, W2_ref[...], preferred_element_type=jnp.float32)
    a = jnp.maximum(a * s2 + t2, 0.0)
    deep = jnp.sum(a * W3_ref[...], axis=1, keepdims=True)
    out_ref[...] = deep + b3_ref[...] + lb_ref[...] + lin_ref[...]


def _mlp(h2d, lin2d, W1, b1, g1, be1, W2, b2, g2, be2, W3r, b3, lbias):
    full = lambda shape: pl.BlockSpec(shape, lambda i: (0, 0))
    return pl.pallas_call(
        _mlp_body,
        grid=(BATCH // MLP_BLK,),
        in_specs=[
            pl.BlockSpec((MLP_BLK, NUM_FIELDS * EMBED_DIM), lambda i: (i, 0)),
            pl.BlockSpec((MLP_BLK, 1), lambda i: (i, 0)),
            full((NUM_FIELDS * EMBED_DIM, 256)),
            full((1, 256)), full((1, 256)), full((1, 256)),
            full((256, 128)),
            full((1, 128)), full((1, 128)), full((1, 128)),
            full((1, 128)),
            full((1, 1)), full((1, 1)),
        ],
        out_specs=pl.BlockSpec((MLP_BLK, 1), lambda i: (i, 0)),
        out_shape=jax.ShapeDtypeStruct((BATCH, 1), jnp.float32),
    )(h2d, lin2d, W1, b1, g1, be1, W2, b2, g2, be2, W3r, b3, lbias)


def kernel(x, emb_W, lin_W, lin_bias, W1, b1, g1, be1, W2, b2, g2, be2,
           W3, b3):
    offs = jnp.arange(NUM_FIELDS, dtype=jnp.int32) * FIELD_DIM
    idx = x.astype(jnp.int32) + offs[None, :]
    idx_emb = idx.reshape(-1)
    emb_lin = _repack(emb_W.T).reshape(TOTAL_VOCAB, EMBED_DIM)
    h, lin_sum = _sc_gather(idx_emb, emb_lin, lin_W.reshape(-1))
    out = _mlp(
        h.reshape(BATCH, NUM_FIELDS * EMBED_DIM),
        lin_sum.reshape(BATCH, 1),
        W1, b1.reshape(1, -1), g1.reshape(1, -1), be1.reshape(1, -1),
        W2, b2.reshape(1, -1), g2.reshape(1, -1), be2.reshape(1, -1),
        W3.reshape(1, -1), b3.reshape(1, 1), lin_bias.reshape(1, 1),
    )
    return jnp.squeeze(out, axis=1)
